# Initial kernel scaffold; baseline (speedup 1.0000x reference)
import jax
import jax.numpy as jnp
from jax.experimental import pallas as pl


def kernel(x, linear_w, embed_w, bias):
    raise NotImplementedError("write your pallas kernel here")


# speedup vs baseline: 1.1534x; 1.1534x over previous
import functools

import numpy as np
import jax
import jax.numpy as jnp
from jax import lax
from jax.experimental import pallas as pl
from jax.experimental.pallas import tpu as pltpu
from jax.experimental.pallas import tpu_sc as plsc

_F = 26
_D = 16
_FIELD_SIZE = 38461
_OFFSETS = np.concatenate(
    [[0], np.cumsum([_FIELD_SIZE] * _F)[:-1]]).astype(np.int32)

_NC = 2
_NS = 16
_NW = _NC * _NS
_C = 64
_IPB = _C * _F
_NCHUNK = _IPB // 128


@functools.cache
def _build(batch):
    assert batch % (_NW * _C) == 0
    b_per_w = batch // _NW
    nblk = b_per_w // _C
    mesh = plsc.VectorSubcoreMesh(core_axis_name="c", subcore_axis_name="s",
                                  num_cores=_NC, num_subcores=_NS)

    def body(x_hbm, lin_hbm, emb_hbm, bias_hbm, off_hbm, out_hbm,
             xv, idxv, offv, biasv, rowsv, linv, outv, sem):
        cid = lax.axis_index("c")
        sid = lax.axis_index("s")
        wid = sid * _NC + cid
        base_row = wid * b_per_w

        pltpu.sync_copy(off_hbm, offv)
        pltpu.sync_copy(bias_hbm, biasv)
        biasvec = biasv[...]
        iota = lax.iota(jnp.int32, 16)

        def blk_body(blk, carry):
            row0 = base_row + blk * _C
            pltpu.sync_copy(x_hbm.at[pl.ds(row0 * _F, _IPB)], xv)
            for j in range(_NCHUNK):
                for k in range(8):
                    sl = pl.ds(j * 128 + k * 16, 16)
                    idxv[j, pl.ds(k * 16, 16)] = xv[sl] + offv[sl]
            copies = []
            for j in range(_NCHUNK):
                copies.append(pltpu.make_async_copy(
                    emb_hbm.at[idxv.at[j]],
                    rowsv.at[pl.ds(j * 128, 128)], sem))
                copies.append(pltpu.make_async_copy(
                    lin_hbm.at[idxv.at[j]],
                    linv.at[pl.ds(j * 128, 128)], sem))
            for c in copies:
                c.start()
            for c in copies:
                c.wait()

            for g in range(_C // 16):
                rbase = iota * _F + g * (16 * _F)
                linsum = jnp.zeros((16,), jnp.float32)
                for f in range(_F):
                    linsum = linsum + plsc.load_gather(linv, [rbase + f])

                def d_body(d, acc):
                    dvec = jnp.zeros((16,), jnp.int32) + d
                    s = jnp.zeros((16,), jnp.float32)
                    sq = jnp.zeros((16,), jnp.float32)
                    for f in range(_F):
                        e = plsc.load_gather(rowsv, [rbase + f, dvec])
                        s = s + e
                        sq = sq + e * e
                    return acc + (s * s - sq)

                acc = lax.fori_loop(0, _D, d_body, jnp.zeros((16,), jnp.float32))
                z = biasvec + linsum + 0.5 * acc
                outv[pl.ds(g * 16, 16)] = 1.0 / (1.0 + jnp.exp(-z))

            pltpu.sync_copy(outv, out_hbm.at[pl.ds(row0, _C)])
            return carry

        lax.fori_loop(0, nblk, blk_body, jnp.int32(0))

    return pl.kernel(
        body,
        out_type=jax.ShapeDtypeStruct((batch,), jnp.float32),
        mesh=mesh,
        scratch_types=[
            pltpu.VMEM((_IPB,), jnp.int32),
            pltpu.VMEM((_NCHUNK, 128), jnp.int32),
            pltpu.VMEM((_IPB,), jnp.int32),
            pltpu.VMEM((16,), jnp.float32),
            pltpu.VMEM((_IPB, _D), jnp.float32),
            pltpu.VMEM((_IPB,), jnp.float32),
            pltpu.VMEM((_C,), jnp.float32),
            pltpu.SemaphoreType.DMA,
        ],
        compiler_params=pltpu.CompilerParams(
            needs_layout_passes=False, use_tc_tiling_on_sc=False),
    )


def kernel(x, linear_w, embed_w, bias):
    batch, nf = x.shape
    assert nf == _F
    x_flat = x.reshape(-1).astype(jnp.int32)
    lin_flat = linear_w.reshape(-1).astype(jnp.float32)
    bias16 = jnp.broadcast_to(bias.reshape(()), (16,)).astype(jnp.float32)
    off_tile = jnp.asarray(np.tile(_OFFSETS, _C))
    out = _build(batch)(x_flat, lin_flat, embed_w, bias16, off_tile)
    return out.reshape(batch, 1)
